# baseline (speedup 1.0000x reference)
import functools

import jax
import jax.numpy as jnp
from jax import lax
from jax.experimental import pallas as pl
from jax.experimental.pallas import tpu as pltpu
from jax.experimental.pallas import tpu_sc as plsc

N_NODES = 10000
N_EDGES = 320000
D_FEAT = 128
N_CLASSES = 16

NC = 2
NS = 16
NW = NC * NS

RPT = 320
OUT_ROWS = NW * RPT
TRASH = RPT
CH = 6400
NCH = N_EDGES // CH
VECS = CH // 16
BLKV = 16
DF = 16
CAPD = 40
GROWS = 16 * DF
PFLAT = 16 * CAPD

ROW_BLK = 1000


def _tc_pool_body(x_ref, wpT_ref, bp_ref, wsT_ref, pooled_ref, xs_ref):
    xb = x_ref[...]
    p = jnp.dot(xb, wpT_ref[...], preferred_element_type=jnp.float32)
    pooled_ref[...] = jnp.maximum(p + bp_ref[...], 0.0)
    xs_ref[...] = jnp.dot(xb, wsT_ref[...], preferred_element_type=jnp.float32)


def _tc_pool(x, wpT, bp, wsT):
    grid = (N_NODES // ROW_BLK,)
    return pl.pallas_call(
        _tc_pool_body,
        grid=grid,
        in_specs=[
            pl.BlockSpec((ROW_BLK, D_FEAT), lambda i: (i, 0)),
            pl.BlockSpec((D_FEAT, D_FEAT), lambda i: (0, 0)),
            pl.BlockSpec((1, D_FEAT), lambda i: (0, 0)),
            pl.BlockSpec((D_FEAT, D_FEAT), lambda i: (0, 0)),
        ],
        out_specs=[
            pl.BlockSpec((ROW_BLK, D_FEAT), lambda i: (i, 0)),
            pl.BlockSpec((ROW_BLK, D_FEAT), lambda i: (i, 0)),
        ],
        out_shape=[
            jax.ShapeDtypeStruct((N_NODES, D_FEAT), jnp.float32),
            jax.ShapeDtypeStruct((N_NODES, D_FEAT), jnp.float32),
        ],
    )(x, wpT, bp, wsT)


def _tc_head_body(xs_ref, hn_ref, wnT_ref, b_ref, wlT_ref, bl_ref, out_ref):
    h = xs_ref[...] + jnp.dot(hn_ref[...], wnT_ref[...],
                              preferred_element_type=jnp.float32) + b_ref[...]
    h = jnp.where(h >= 0.0, h, 0.01 * h)
    z = jnp.dot(h, wlT_ref[...], preferred_element_type=jnp.float32) + bl_ref[...]
    out_ref[...] = jax.nn.sigmoid(z)


def _tc_head(xs, hn, wnT, b, wlT, bl):
    grid = (N_NODES // ROW_BLK,)
    return pl.pallas_call(
        _tc_head_body,
        grid=grid,
        in_specs=[
            pl.BlockSpec((ROW_BLK, D_FEAT), lambda i: (i, 0)),
            pl.BlockSpec((ROW_BLK, D_FEAT), lambda i: (i, 0)),
            pl.BlockSpec((D_FEAT, D_FEAT), lambda i: (0, 0)),
            pl.BlockSpec((1, D_FEAT), lambda i: (0, 0)),
            pl.BlockSpec((D_FEAT, N_CLASSES), lambda i: (0, 0)),
            pl.BlockSpec((1, N_CLASSES), lambda i: (0, 0)),
        ],
        out_specs=pl.BlockSpec((ROW_BLK, N_CLASSES), lambda i: (i, 0)),
        out_shape=jax.ShapeDtypeStruct((N_NODES, N_CLASSES), jnp.float32),
    )(xs, hn, wnT, b, wlT, bl)


def _sc_body(pooled_hbm, src_hbm, dst_hbm, out_hbm,
             acc, dst_buf, src_buf, dst_buf2, src_buf2,
             pend_src, pend_ld, gidx_a, gidx_b, gld, rows_a, rows_b,
             out_ref, sem, semb, semd, sems):
    wid = lax.axis_index("s") * NC + lax.axis_index("c")
    lo = wid * RPT
    hi = lo + RPT

    zero16 = jnp.zeros((16,), jnp.float32)

    def _zrow(r, _):
        for f in range(8):
            acc[r, pl.ds(16 * f, 16)] = zero16
        return 0
    lax.fori_loop(0, RPT + 1, _zrow, 0)
    out_ref[0] = 0

    ztrash = jnp.full((16,), TRASH, jnp.int32)
    zsrc = jnp.zeros((16,), jnp.int32)
    for k in range(CAPD):
        pend_src[pl.ds(16 * k, 16)] = zsrc
        pend_ld[pl.ds(16 * k, 16)] = ztrash

    def _accum_group(rows_ref, row_base, ld_ref, ld_base):
        ldv = ld_ref[pl.ds(ld_base, 16)]
        for i in range(16):
            ld = ldv[i]
            for f in range(8):
                sl = pl.ds(16 * f, 16)
                acc[ld, sl] = jnp.maximum(acc[ld, sl], rows_ref[row_base + i, sl])

    def _wait_and_accum_inflight():
        @pl.when(out_ref[0] == 1)
        def _():
            pltpu.make_async_copy(pooled_hbm.at[gidx_a], rows_a, sem).wait()
            pltpu.make_async_copy(pooled_hbm.at[gidx_b], rows_b, semb).wait()

            def _agrp(g, _):
                _accum_group(rows_a, g * 16, gld, g * 16)
                _accum_group(rows_b, g * 16, gld, 128 + g * 16)
                return 0
            lax.fori_loop(0, 8, _agrp, 0)
            out_ref[0] = 0

    def _flush_pend(dv):
        _wait_and_accum_inflight()
        for k in range(8):
            gidx_a[pl.ds(16 * k, 16)] = pend_src[pl.ds(16 * k, 16)]
            gld[pl.ds(16 * k, 16)] = pend_ld[pl.ds(16 * k, 16)]
        for k in range(8):
            gidx_b[pl.ds(16 * k, 16)] = pend_src[pl.ds(128 + 16 * k, 16)]
            gld[pl.ds(128 + 16 * k, 16)] = pend_ld[pl.ds(128 + 16 * k, 16)]
        for k in range(CAPD - DF):
            pend_src[pl.ds(16 * k, 16)] = pend_src[pl.ds(16 * (k + DF), 16)]
            pend_ld[pl.ds(16 * k, 16)] = pend_ld[pl.ds(16 * (k + DF), 16)]
        pltpu.async_copy(pooled_hbm.at[gidx_a], rows_a, sem)
        pltpu.async_copy(pooled_hbm.at[gidx_b], rows_b, semb)
        out_ref[0] = 1
        return jnp.maximum(dv - DF, 0)

    iota16 = lax.iota(jnp.int32, 16)
    urpt = jnp.uint32(RPT)

    def _scan_buf(db, sb, depthv):
        def _blk(b, dv):
            for u in range(BLKV):
                e0 = (b * BLKV + u) * 16
                d = db[pl.ds(e0, 16)]
                s = sb[pl.ds(e0, 16)]
                ud = d - lo
                m = plsc.bitcast(ud, jnp.uint32) < urpt
                pos = iota16 + dv * 16
                plsc.store_scatter(pend_src, [pos], s, mask=m)
                plsc.store_scatter(pend_ld, [pos], ud, mask=m)
                dv = dv + jnp.where(m, 1, 0)
            pa = plsc.all_reduce_population_count(dv >= DF)
            pb = plsc.all_reduce_population_count(dv >= CAPD - BLKV)
            full = (pa[0] == 16) | (pb[0] > 0)
            return lax.cond(full, _flush_pend, lambda x: x, dv)

        return lax.fori_loop(0, VECS // BLKV, _blk, depthv)

    def _start_chunk(ci, db, sb):
        pltpu.async_copy(dst_hbm.at[pl.ds(ci * CH, CH)], db, semd)
        pltpu.async_copy(src_hbm.at[pl.ds(ci * CH, CH)], sb, sems)

    def _wait_chunk(ci, db, sb):
        pltpu.make_async_copy(dst_hbm.at[pl.ds(ci * CH, CH)], db, semd).wait()
        pltpu.make_async_copy(src_hbm.at[pl.ds(ci * CH, CH)], sb, sems).wait()

    _start_chunk(0, dst_buf, src_buf)

    def _outer(g, cv):
        for u, (db, sb) in ((0, (dst_buf, src_buf)), (1, (dst_buf2, src_buf2))):
            ci = 2 * g + u
            _wait_chunk(ci, db, sb)
            nb = (dst_buf2, src_buf2) if u == 0 else (dst_buf, src_buf)

            @pl.when(ci + 1 < NCH)
            def _pref():
                _start_chunk(ci + 1, nb[0], nb[1])
            cv = _scan_buf(db, sb, cv)
        return cv

    lax.fori_loop(0, NCH // 2, _outer, jnp.zeros((16,), jnp.int32))

    _wait_and_accum_inflight()

    for bi in range(PFLAT // 128):
        for k in range(8):
            gidx_a[pl.ds(16 * k, 16)] = pend_src[pl.ds(128 * bi + 16 * k, 16)]
        pltpu.async_copy(pooled_hbm.at[gidx_a], rows_a, sem).wait()

        def _dgrp(g, _):
            _accum_group(rows_a, g * 16, pend_ld, 128 * bi + g * 16)
            return 0
        lax.fori_loop(0, 8, _dgrp, 0)

    pltpu.sync_copy(acc.at[pl.ds(0, RPT)], out_hbm.at[pl.ds(lo, RPT)])


@functools.partial(
    pl.kernel,
    out_type=jax.ShapeDtypeStruct((OUT_ROWS, D_FEAT), jnp.float32),
    mesh=plsc.VectorSubcoreMesh(core_axis_name="c", subcore_axis_name="s"),
    compiler_params=pltpu.CompilerParams(needs_layout_passes=False),
    scratch_types=[
        pltpu.VMEM((RPT + 1, D_FEAT), jnp.float32),
        pltpu.VMEM((CH,), jnp.int32),
        pltpu.VMEM((CH,), jnp.int32),
        pltpu.VMEM((CH,), jnp.int32),
        pltpu.VMEM((CH,), jnp.int32),
        pltpu.VMEM((PFLAT,), jnp.int32),
        pltpu.VMEM((PFLAT,), jnp.int32),
        pltpu.VMEM((128,), jnp.int32),
        pltpu.VMEM((128,), jnp.int32),
        pltpu.VMEM((GROWS,), jnp.int32),
        pltpu.VMEM((128, D_FEAT), jnp.float32),
        pltpu.VMEM((128, D_FEAT), jnp.float32),
        pltpu.SMEM((1,), jnp.int32),
        pltpu.SemaphoreType.DMA,
        pltpu.SemaphoreType.DMA,
        pltpu.SemaphoreType.DMA,
        pltpu.SemaphoreType.DMA,
    ],
)
def _sc_segmax(pooled_hbm, src_hbm, dst_hbm, out_hbm,
               acc, dst_buf, src_buf, dst_buf2, src_buf2,
               pend_src, pend_ld, gidx_a, gidx_b, gld, rows_a, rows_b,
             out_ref, sem, semb, semd, sems):
    _sc_body(pooled_hbm, src_hbm, dst_hbm, out_hbm,
             acc, dst_buf, src_buf, dst_buf2, src_buf2,
             pend_src, pend_ld, gidx_a, gidx_b, gld, rows_a, rows_b,
             out_ref, sem, semb, semd, sems)


def kernel(x, edge_index, W_pool, b_pool, W_self, W_neigh, bias, W_lin, b_lin):
    src = edge_index[0]
    dst = edge_index[1]
    pooled, xs = _tc_pool(x, W_pool.T, b_pool.reshape(1, -1), W_self.T)
    hn_pad = _sc_segmax(pooled, src, dst)
    hn = hn_pad[:N_NODES]
    return _tc_head(xs, hn, W_neigh.T, bias.reshape(1, -1),
                    W_lin.T, b_lin.reshape(1, -1))


# speedup vs baseline: 1.5558x; 1.5558x over previous
import functools

import jax
import jax.numpy as jnp
from jax import lax
from jax.experimental import pallas as pl
from jax.experimental.pallas import tpu as pltpu
from jax.experimental.pallas import tpu_sc as plsc

N_NODES = 10000
N_EDGES = 320000
D_FEAT = 128
N_CLASSES = 16

NC = 2
NS = 16
NW = NC * NS

RPT = 320
OUT_ROWS = NW * RPT
TRASH = RPT
CH = 6400
NCH = N_EDGES // CH
VECS = CH // 16
BLKV = 8
DF = 8
CAPD = 32
GROWS = 16 * DF
PFLAT = 16 * CAPD

ROW_BLK = 1000


def _tc_pool_body(x_ref, wpT_ref, bp_ref, wsT_ref, pooled_ref, xs_ref):
    xb = x_ref[...]
    p = jnp.dot(xb, wpT_ref[...], preferred_element_type=jnp.float32)
    pooled_ref[...] = jnp.maximum(p + bp_ref[...], 0.0)
    xs_ref[...] = jnp.dot(xb, wsT_ref[...], preferred_element_type=jnp.float32)


def _tc_pool(x, wpT, bp, wsT):
    grid = (N_NODES // ROW_BLK,)
    return pl.pallas_call(
        _tc_pool_body,
        grid=grid,
        in_specs=[
            pl.BlockSpec((ROW_BLK, D_FEAT), lambda i: (i, 0)),
            pl.BlockSpec((D_FEAT, D_FEAT), lambda i: (0, 0)),
            pl.BlockSpec((1, D_FEAT), lambda i: (0, 0)),
            pl.BlockSpec((D_FEAT, D_FEAT), lambda i: (0, 0)),
        ],
        out_specs=[
            pl.BlockSpec((ROW_BLK, D_FEAT), lambda i: (i, 0)),
            pl.BlockSpec((ROW_BLK, D_FEAT), lambda i: (i, 0)),
        ],
        out_shape=[
            jax.ShapeDtypeStruct((N_NODES, D_FEAT), jnp.float32),
            jax.ShapeDtypeStruct((N_NODES, D_FEAT), jnp.float32),
        ],
    )(x, wpT, bp, wsT)


def _tc_head_body(xs_ref, hn_ref, wnT_ref, b_ref, wlT_ref, bl_ref, out_ref):
    h = xs_ref[...] + jnp.dot(hn_ref[...], wnT_ref[...],
                              preferred_element_type=jnp.float32) + b_ref[...]
    h = jnp.where(h >= 0.0, h, 0.01 * h)
    z = jnp.dot(h, wlT_ref[...], preferred_element_type=jnp.float32) + bl_ref[...]
    out_ref[...] = jax.nn.sigmoid(z)


def _tc_head(xs, hn, wnT, b, wlT, bl):
    grid = (N_NODES // ROW_BLK,)
    return pl.pallas_call(
        _tc_head_body,
        grid=grid,
        in_specs=[
            pl.BlockSpec((ROW_BLK, D_FEAT), lambda i: (i, 0)),
            pl.BlockSpec((ROW_BLK, D_FEAT), lambda i: (i, 0)),
            pl.BlockSpec((D_FEAT, D_FEAT), lambda i: (0, 0)),
            pl.BlockSpec((1, D_FEAT), lambda i: (0, 0)),
            pl.BlockSpec((D_FEAT, N_CLASSES), lambda i: (0, 0)),
            pl.BlockSpec((1, N_CLASSES), lambda i: (0, 0)),
        ],
        out_specs=pl.BlockSpec((ROW_BLK, N_CLASSES), lambda i: (i, 0)),
        out_shape=jax.ShapeDtypeStruct((N_NODES, N_CLASSES), jnp.float32),
    )(xs, hn, wnT, b, wlT, bl)


def _sc_body(pooled_hbm, src_hbm, dst_hbm, out_hbm,
             acc, dst_buf, src_buf, dst_buf2, src_buf2,
             pend_src, pend_ld, gidx_a, gidx_b, gld, rows_a, rows_b,
             out_ref, sem, semb, semd, sems):
    wid = lax.axis_index("s") * NC + lax.axis_index("c")
    lo = wid * RPT
    hi = lo + RPT

    zero16 = jnp.zeros((16,), jnp.float32)

    def _zrow(r, _):
        for f in range(8):
            acc[r, pl.ds(16 * f, 16)] = zero16
        return 0
    lax.fori_loop(0, RPT + 1, _zrow, 0)
    out_ref[0] = 0

    ztrash = jnp.full((16,), TRASH, jnp.int32)
    zsrc = jnp.zeros((16,), jnp.int32)
    for k in range(CAPD):
        pend_src[pl.ds(16 * k, 16)] = zsrc
        pend_ld[pl.ds(16 * k, 16)] = ztrash

    def _accum_group(rows_ref, row_base, ld_ref, ld_base):
        ldv = ld_ref[pl.ds(ld_base, 16)]
        for i in range(16):
            ld = ldv[i]
            for f in range(8):
                sl = pl.ds(16 * f, 16)
                acc[ld, sl] = jnp.maximum(acc[ld, sl], rows_ref[row_base + i, sl])

    def _wait_and_accum_inflight():
        @pl.when(out_ref[0] == 1)
        def _():
            pltpu.make_async_copy(pooled_hbm.at[gidx_a], rows_a, sem).wait()

            def _agrp(g, _):
                _accum_group(rows_a, g * 16, gld, g * 16)
                return 0
            lax.fori_loop(0, 8, _agrp, 0)
            out_ref[0] = 0

    def _flush_pend(dv):
        _wait_and_accum_inflight()
        for k in range(8):
            gidx_a[pl.ds(16 * k, 16)] = pend_src[pl.ds(16 * k, 16)]
            gld[pl.ds(16 * k, 16)] = pend_ld[pl.ds(16 * k, 16)]
        for k in range(CAPD - DF):
            pend_src[pl.ds(16 * k, 16)] = pend_src[pl.ds(16 * (k + DF), 16)]
            pend_ld[pl.ds(16 * k, 16)] = pend_ld[pl.ds(16 * (k + DF), 16)]
        pltpu.async_copy(pooled_hbm.at[gidx_a], rows_a, sem)
        out_ref[0] = 1
        return jnp.maximum(dv - DF, 0)

    iota16 = lax.iota(jnp.int32, 16)
    urpt = jnp.uint32(RPT)

    def _scan_buf(db, sb, depthv):
        def _blk(b, dv):
            for u in range(BLKV):
                e0 = (b * BLKV + u) * 16
                d = db[pl.ds(e0, 16)]
                s = sb[pl.ds(e0, 16)]
                ud = d - lo
                m = plsc.bitcast(ud, jnp.uint32) < urpt
                pos = iota16 + dv * 16
                plsc.store_scatter(pend_src, [pos], s, mask=m)
                plsc.store_scatter(pend_ld, [pos], ud, mask=m)
                dv = dv + jnp.where(m, 1, 0)
            pa = plsc.all_reduce_population_count(dv >= DF)
            pb = plsc.all_reduce_population_count(dv >= CAPD - BLKV)
            full = (pa[0] == 16) | (pb[0] > 0)
            return lax.cond(full, _flush_pend, lambda x: x, dv)

        return lax.fori_loop(0, VECS // BLKV, _blk, depthv)

    def _start_chunk(ci, db, sb):
        pltpu.async_copy(dst_hbm.at[pl.ds(ci * CH, CH)], db, semd)
        pltpu.async_copy(src_hbm.at[pl.ds(ci * CH, CH)], sb, sems)

    def _wait_chunk(ci, db, sb):
        pltpu.make_async_copy(dst_hbm.at[pl.ds(ci * CH, CH)], db, semd).wait()
        pltpu.make_async_copy(src_hbm.at[pl.ds(ci * CH, CH)], sb, sems).wait()

    _start_chunk(0, dst_buf, src_buf)

    def _outer(g, cv):
        for u, (db, sb) in ((0, (dst_buf, src_buf)), (1, (dst_buf2, src_buf2))):
            ci = 2 * g + u
            _wait_chunk(ci, db, sb)
            nb = (dst_buf2, src_buf2) if u == 0 else (dst_buf, src_buf)

            @pl.when(ci + 1 < NCH)
            def _pref():
                _start_chunk(ci + 1, nb[0], nb[1])
            cv = _scan_buf(db, sb, cv)
        return cv

    lax.fori_loop(0, NCH // 2, _outer, jnp.zeros((16,), jnp.int32))

    _wait_and_accum_inflight()

    for bi in range(PFLAT // 128):
        for k in range(8):
            gidx_a[pl.ds(16 * k, 16)] = pend_src[pl.ds(128 * bi + 16 * k, 16)]
        pltpu.async_copy(pooled_hbm.at[gidx_a], rows_a, sem).wait()

        def _dgrp(g, _):
            _accum_group(rows_a, g * 16, pend_ld, 128 * bi + g * 16)
            return 0
        lax.fori_loop(0, 8, _dgrp, 0)

    pltpu.sync_copy(acc.at[pl.ds(0, RPT)], out_hbm.at[pl.ds(lo, RPT)])


@functools.partial(
    pl.kernel,
    out_type=jax.ShapeDtypeStruct((OUT_ROWS, D_FEAT), jnp.float32),
    mesh=plsc.VectorSubcoreMesh(core_axis_name="c", subcore_axis_name="s"),
    compiler_params=pltpu.CompilerParams(needs_layout_passes=False),
    scratch_types=[
        pltpu.VMEM((RPT + 1, D_FEAT), jnp.float32),
        pltpu.VMEM((CH,), jnp.int32),
        pltpu.VMEM((CH,), jnp.int32),
        pltpu.VMEM((CH,), jnp.int32),
        pltpu.VMEM((CH,), jnp.int32),
        pltpu.VMEM((PFLAT,), jnp.int32),
        pltpu.VMEM((PFLAT,), jnp.int32),
        pltpu.VMEM((128,), jnp.int32),
        pltpu.VMEM((128,), jnp.int32),
        pltpu.VMEM((GROWS,), jnp.int32),
        pltpu.VMEM((128, D_FEAT), jnp.float32),
        pltpu.VMEM((16, D_FEAT), jnp.float32),
        pltpu.SMEM((1,), jnp.int32),
        pltpu.SemaphoreType.DMA,
        pltpu.SemaphoreType.DMA,
        pltpu.SemaphoreType.DMA,
        pltpu.SemaphoreType.DMA,
    ],
)
def _sc_segmax(pooled_hbm, src_hbm, dst_hbm, out_hbm,
               acc, dst_buf, src_buf, dst_buf2, src_buf2,
               pend_src, pend_ld, gidx_a, gidx_b, gld, rows_a, rows_b,
             out_ref, sem, semb, semd, sems):
    _sc_body(pooled_hbm, src_hbm, dst_hbm, out_hbm,
             acc, dst_buf, src_buf, dst_buf2, src_buf2,
             pend_src, pend_ld, gidx_a, gidx_b, gld, rows_a, rows_b,
             out_ref, sem, semb, semd, sems)


def kernel(x, edge_index, W_pool, b_pool, W_self, W_neigh, bias, W_lin, b_lin):
    src = edge_index[0]
    dst = edge_index[1]
    pooled, xs = _tc_pool(x, W_pool.T, b_pool.reshape(1, -1), W_self.T)
    hn_pad = _sc_segmax(pooled, src, dst)
    hn = hn_pad[:N_NODES]
    return _tc_head(xs, hn, W_neigh.T, bias.reshape(1, -1),
                    W_lin.T, b_lin.reshape(1, -1))


# speedup vs baseline: 1.5578x; 1.0013x over previous
import functools

import jax
import jax.numpy as jnp
from jax import lax
from jax.experimental import pallas as pl
from jax.experimental.pallas import tpu as pltpu
from jax.experimental.pallas import tpu_sc as plsc

N_NODES = 10000
N_EDGES = 320000
D_FEAT = 128
N_CLASSES = 16

NC = 2
NS = 16
NW = NC * NS

RPT = 320
OUT_ROWS = NW * RPT
TRASH = RPT
CH = 6400
NCH = N_EDGES // CH
VECS = CH // 16
BLKV = 8
DF = 8
CAPD = 32
GROWS = 16 * DF
PFLAT = 16 * CAPD

ROW_BLK = 1000


def _tc_pool_body(x_ref, wpT_ref, bp_ref, wsT_ref, pooled_ref, xs_ref):
    xb = x_ref[...]
    p = jnp.dot(xb, wpT_ref[...], preferred_element_type=jnp.float32)
    pooled_ref[...] = jnp.maximum(p + bp_ref[...], 0.0)
    xs_ref[...] = jnp.dot(xb, wsT_ref[...], preferred_element_type=jnp.float32)


def _tc_pool(x, wpT, bp, wsT):
    grid = (N_NODES // ROW_BLK,)
    return pl.pallas_call(
        _tc_pool_body,
        grid=grid,
        in_specs=[
            pl.BlockSpec((ROW_BLK, D_FEAT), lambda i: (i, 0)),
            pl.BlockSpec((D_FEAT, D_FEAT), lambda i: (0, 0)),
            pl.BlockSpec((1, D_FEAT), lambda i: (0, 0)),
            pl.BlockSpec((D_FEAT, D_FEAT), lambda i: (0, 0)),
        ],
        out_specs=[
            pl.BlockSpec((ROW_BLK, D_FEAT), lambda i: (i, 0)),
            pl.BlockSpec((ROW_BLK, D_FEAT), lambda i: (i, 0)),
        ],
        out_shape=[
            jax.ShapeDtypeStruct((N_NODES, D_FEAT), jnp.float32),
            jax.ShapeDtypeStruct((N_NODES, D_FEAT), jnp.float32),
        ],
    )(x, wpT, bp, wsT)


def _tc_head_body(xs_ref, hn_ref, wnT_ref, b_ref, wlT_ref, bl_ref, out_ref):
    h = xs_ref[...] + jnp.dot(hn_ref[...], wnT_ref[...],
                              preferred_element_type=jnp.float32) + b_ref[...]
    h = jnp.where(h >= 0.0, h, 0.01 * h)
    z = jnp.dot(h, wlT_ref[...], preferred_element_type=jnp.float32) + bl_ref[...]
    out_ref[...] = jax.nn.sigmoid(z)


def _tc_head(xs, hn, wnT, b, wlT, bl):
    grid = (N_NODES // ROW_BLK,)
    return pl.pallas_call(
        _tc_head_body,
        grid=grid,
        in_specs=[
            pl.BlockSpec((ROW_BLK, D_FEAT), lambda i: (i, 0)),
            pl.BlockSpec((ROW_BLK, D_FEAT), lambda i: (i, 0)),
            pl.BlockSpec((D_FEAT, D_FEAT), lambda i: (0, 0)),
            pl.BlockSpec((1, D_FEAT), lambda i: (0, 0)),
            pl.BlockSpec((D_FEAT, N_CLASSES), lambda i: (0, 0)),
            pl.BlockSpec((1, N_CLASSES), lambda i: (0, 0)),
        ],
        out_specs=pl.BlockSpec((ROW_BLK, N_CLASSES), lambda i: (i, 0)),
        out_shape=jax.ShapeDtypeStruct((N_NODES, N_CLASSES), jnp.float32),
    )(xs, hn, wnT, b, wlT, bl)


def _sc_body(pooled_hbm, src_hbm, dst_hbm, out_hbm,
             acc, dst_buf, src_buf, dst_buf2, src_buf2,
             pend_src, pend_ld, gidx_a, gidx_b, gld, rows_a, rows_b,
             out_ref, sem, semb, semd, sems):
    wid = lax.axis_index("s") * NC + lax.axis_index("c")
    lo = wid * RPT
    hi = lo + RPT

    zero16 = jnp.zeros((16,), jnp.float32)

    def _zrow(r, _):
        for f in range(8):
            acc[r, pl.ds(16 * f, 16)] = zero16
        return 0
    lax.fori_loop(0, RPT + 1, _zrow, 0)
    out_ref[0] = 0

    ztrash = jnp.full((16,), TRASH, jnp.int32)
    zsrc = jnp.zeros((16,), jnp.int32)
    for k in range(CAPD):
        pend_src[pl.ds(16 * k, 16)] = zsrc
        pend_ld[pl.ds(16 * k, 16)] = ztrash

    def _accum_group(rows_ref, row_base, ld_ref, ld_base):
        ldv = ld_ref[pl.ds(ld_base, 16)]
        for i in range(16):
            ld = ldv[i]
            for f in range(8):
                sl = pl.ds(16 * f, 16)
                acc[ld, sl] = jnp.maximum(acc[ld, sl], rows_ref[row_base + i, sl])

    def _wait_and_accum_inflight():
        @pl.when(out_ref[0] == 1)
        def _():
            pltpu.make_async_copy(pooled_hbm.at[gidx_a], rows_a, sem).wait()

            def _agrp(g, _):
                _accum_group(rows_a, g * 16, gld, g * 16)
                return 0
            lax.fori_loop(0, 8, _agrp, 0)
            out_ref[0] = 0

    def _flush_pend(dv):
        _wait_and_accum_inflight()
        for k in range(8):
            gidx_a[pl.ds(16 * k, 16)] = pend_src[pl.ds(16 * k, 16)]
            gld[pl.ds(16 * k, 16)] = pend_ld[pl.ds(16 * k, 16)]
        for k in range(CAPD - DF):
            pend_src[pl.ds(16 * k, 16)] = pend_src[pl.ds(16 * (k + DF), 16)]
            pend_ld[pl.ds(16 * k, 16)] = pend_ld[pl.ds(16 * (k + DF), 16)]
        pltpu.async_copy(pooled_hbm.at[gidx_a], rows_a, sem)
        out_ref[0] = 1
        return jnp.maximum(dv - DF, 0)

    iota16 = lax.iota(jnp.int32, 16)
    urpt = jnp.uint32(RPT)

    def _scan_buf(db, sb, depthv):
        def _blk(b, dv):
            for u in range(BLKV):
                e0 = (b * BLKV + u) * 16
                d = db[pl.ds(e0, 16)]
                s = sb[pl.ds(e0, 16)]
                ud = d - lo
                m = (ud >= 0) & (ud < RPT)
                pos = iota16 + dv * 16
                plsc.store_scatter(pend_src, [pos], s, mask=m)
                plsc.store_scatter(pend_ld, [pos], ud, mask=m)
                dv = dv + jnp.where(m, 1, 0)
            pa = plsc.all_reduce_population_count(dv >= DF)
            pb = plsc.all_reduce_population_count(dv >= CAPD - BLKV)
            full = (pa[0] == 16) | (pb[0] > 0)
            return lax.cond(full, _flush_pend, lambda x: x, dv)

        return lax.fori_loop(0, VECS // BLKV, _blk, depthv)

    def _start_chunk(ci, db, sb):
        pltpu.async_copy(dst_hbm.at[pl.ds(ci * CH, CH)], db, semd)
        pltpu.async_copy(src_hbm.at[pl.ds(ci * CH, CH)], sb, sems)

    def _wait_chunk(ci, db, sb):
        pltpu.make_async_copy(dst_hbm.at[pl.ds(ci * CH, CH)], db, semd).wait()
        pltpu.make_async_copy(src_hbm.at[pl.ds(ci * CH, CH)], sb, sems).wait()

    _start_chunk(0, dst_buf, src_buf)

    def _outer(g, cv):
        for u, (db, sb) in ((0, (dst_buf, src_buf)), (1, (dst_buf2, src_buf2))):
            ci = 2 * g + u
            _wait_chunk(ci, db, sb)
            nb = (dst_buf2, src_buf2) if u == 0 else (dst_buf, src_buf)

            @pl.when(ci + 1 < NCH)
            def _pref():
                _start_chunk(ci + 1, nb[0], nb[1])
            cv = _scan_buf(db, sb, cv)
        return cv

    lax.fori_loop(0, NCH // 2, _outer, jnp.zeros((16,), jnp.int32))

    _wait_and_accum_inflight()

    for bi in range(PFLAT // 128):
        for k in range(8):
            gidx_a[pl.ds(16 * k, 16)] = pend_src[pl.ds(128 * bi + 16 * k, 16)]
        pltpu.async_copy(pooled_hbm.at[gidx_a], rows_a, sem).wait()

        def _dgrp(g, _):
            _accum_group(rows_a, g * 16, pend_ld, 128 * bi + g * 16)
            return 0
        lax.fori_loop(0, 8, _dgrp, 0)

    pltpu.sync_copy(acc.at[pl.ds(0, RPT)], out_hbm.at[pl.ds(lo, RPT)])


@functools.partial(
    pl.kernel,
    out_type=jax.ShapeDtypeStruct((OUT_ROWS, D_FEAT), jnp.float32),
    mesh=plsc.VectorSubcoreMesh(core_axis_name="c", subcore_axis_name="s"),
    compiler_params=pltpu.CompilerParams(needs_layout_passes=False),
    scratch_types=[
        pltpu.VMEM((RPT + 1, D_FEAT), jnp.float32),
        pltpu.VMEM((CH,), jnp.int32),
        pltpu.VMEM((CH,), jnp.int32),
        pltpu.VMEM((CH,), jnp.int32),
        pltpu.VMEM((CH,), jnp.int32),
        pltpu.VMEM((PFLAT,), jnp.int32),
        pltpu.VMEM((PFLAT,), jnp.int32),
        pltpu.VMEM((128,), jnp.int32),
        pltpu.VMEM((128,), jnp.int32),
        pltpu.VMEM((GROWS,), jnp.int32),
        pltpu.VMEM((128, D_FEAT), jnp.float32),
        pltpu.VMEM((16, D_FEAT), jnp.float32),
        pltpu.SMEM((1,), jnp.int32),
        pltpu.SemaphoreType.DMA,
        pltpu.SemaphoreType.DMA,
        pltpu.SemaphoreType.DMA,
        pltpu.SemaphoreType.DMA,
    ],
)
def _sc_segmax(pooled_hbm, src_hbm, dst_hbm, out_hbm,
               acc, dst_buf, src_buf, dst_buf2, src_buf2,
               pend_src, pend_ld, gidx_a, gidx_b, gld, rows_a, rows_b,
             out_ref, sem, semb, semd, sems):
    _sc_body(pooled_hbm, src_hbm, dst_hbm, out_hbm,
             acc, dst_buf, src_buf, dst_buf2, src_buf2,
             pend_src, pend_ld, gidx_a, gidx_b, gld, rows_a, rows_b,
             out_ref, sem, semb, semd, sems)


def kernel(x, edge_index, W_pool, b_pool, W_self, W_neigh, bias, W_lin, b_lin):
    src = edge_index[0]
    dst = edge_index[1]
    pooled, xs = _tc_pool(x, W_pool.T, b_pool.reshape(1, -1), W_self.T)
    hn_pad = _sc_segmax(pooled, src, dst)
    hn = hn_pad[:N_NODES]
    return _tc_head(xs, hn, W_neigh.T, bias.reshape(1, -1),
                    W_lin.T, b_lin.reshape(1, -1))


# speedup vs baseline: 4.0991x; 2.6313x over previous
import functools

import jax
import jax.numpy as jnp
from jax import lax
from jax.experimental import pallas as pl
from jax.experimental.pallas import tpu as pltpu
from jax.experimental.pallas import tpu_sc as plsc

N_NODES = 10000
N_EDGES = 320000
D_FEAT = 128
N_CLASSES = 16

NC = 2
NS = 16
NW = NC * NS

RPT = 320
OUT_ROWS = NW * RPT
TRASH = RPT
CH = 6400
NCH = N_EDGES // CH
VECS = CH // 16
BLK = 8
G = 128
PEND = 272

ROW_BLK = 1000
HEAD_BLK = 400


def _tc_pool_body(x_ref, wpT_ref, bp_ref, wsT_ref, pooled_ref, xs_ref):
    xb = x_ref[...]
    p = jnp.dot(xb, wpT_ref[...], preferred_element_type=jnp.float32)
    pooled_ref[...] = jnp.maximum(p + bp_ref[...], 0.0)
    xs_ref[...] = jnp.dot(xb, wsT_ref[...], preferred_element_type=jnp.float32)


def _tc_pool(x, wpT, bp, wsT):
    grid = (N_NODES // ROW_BLK,)
    return pl.pallas_call(
        _tc_pool_body,
        grid=grid,
        in_specs=[
            pl.BlockSpec((ROW_BLK, D_FEAT), lambda i: (i, 0)),
            pl.BlockSpec((D_FEAT, D_FEAT), lambda i: (0, 0)),
            pl.BlockSpec((1, D_FEAT), lambda i: (0, 0)),
            pl.BlockSpec((D_FEAT, D_FEAT), lambda i: (0, 0)),
        ],
        out_specs=[
            pl.BlockSpec((ROW_BLK, D_FEAT), lambda i: (i, 0)),
            pl.BlockSpec((ROW_BLK, D_FEAT), lambda i: (i, 0)),
        ],
        out_shape=[
            jax.ShapeDtypeStruct((N_NODES, D_FEAT), jnp.float32),
            jax.ShapeDtypeStruct((N_NODES, D_FEAT), jnp.float32),
        ],
    )(x, wpT, bp, wsT)


def _tc_head_body(xs_ref, hn_ref, wnT_ref, b_ref, wlT_ref, bl_ref, out_ref):
    h = xs_ref[...] + jnp.dot(hn_ref[...], wnT_ref[...],
                              preferred_element_type=jnp.float32) + b_ref[...]
    h = jnp.where(h >= 0.0, h, 0.01 * h)
    z = jnp.dot(h, wlT_ref[...], preferred_element_type=jnp.float32) + bl_ref[...]
    out_ref[...] = jax.nn.sigmoid(z)


def _tc_head(xs, hn, wnT, b, wlT, bl):
    grid = (N_NODES // HEAD_BLK,)
    return pl.pallas_call(
        _tc_head_body,
        grid=grid,
        in_specs=[
            pl.BlockSpec((HEAD_BLK, D_FEAT), lambda i: (i, 0)),
            pl.BlockSpec((HEAD_BLK, D_FEAT), lambda i: (i, 0)),
            pl.BlockSpec((D_FEAT, D_FEAT), lambda i: (0, 0)),
            pl.BlockSpec((1, D_FEAT), lambda i: (0, 0)),
            pl.BlockSpec((D_FEAT, N_CLASSES), lambda i: (0, 0)),
            pl.BlockSpec((1, N_CLASSES), lambda i: (0, 0)),
        ],
        out_specs=pl.BlockSpec((HEAD_BLK, N_CLASSES), lambda i: (i, 0)),
        out_shape=jax.ShapeDtypeStruct((N_NODES, N_CLASSES), jnp.float32),
    )(xs, hn, wnT, b, wlT, bl)


def _sc_body(pooled_hbm, src_hbm, dst_hbm, out_hbm,
             acc, dst_buf, src_buf, dst_buf2, src_buf2,
             pend_src, pend_ld, gidx, gld, rows_buf, out_ref, sem, semd, sems):
    wid = lax.axis_index("s") * NC + lax.axis_index("c")
    lo = wid * RPT
    hi = lo + RPT

    zero16 = jnp.zeros((16,), jnp.float32)

    def _zrow(r, _):
        for f in range(8):
            acc[r, pl.ds(16 * f, 16)] = zero16
        return 0
    lax.fori_loop(0, RPT + 1, _zrow, 0)
    out_ref[0] = 0

    def _accum_group(rows_ref, row_base, ld_ref, ld_base):
        ldv = ld_ref[pl.ds(ld_base, 16)]
        for i in range(16):
            ld = ldv[i]
            for f in range(8):
                sl = pl.ds(16 * f, 16)
                acc[ld, sl] = jnp.maximum(acc[ld, sl], rows_ref[row_base + i, sl])

    def _wait_and_accum_inflight():
        @pl.when(out_ref[0] == 1)
        def _():
            pltpu.make_async_copy(pooled_hbm.at[gidx], rows_buf, sem).wait()

            def _agrp(g, _):
                _accum_group(rows_buf, g * 16, gld, g * 16)
                return 0
            lax.fori_loop(0, G // 16, _agrp, 0)
            out_ref[0] = 0

    def _flush_pend(cv):
        _wait_and_accum_inflight()
        for k in range(G // 16):
            gidx[pl.ds(16 * k, 16)] = pend_src[pl.ds(16 * k, 16)]
            gld[pl.ds(16 * k, 16)] = pend_ld[pl.ds(16 * k, 16)]
        for k in range(BLK):
            pend_src[pl.ds(16 * k, 16)] = pend_src[pl.ds(G + 16 * k, 16)]
            pend_ld[pl.ds(16 * k, 16)] = pend_ld[pl.ds(G + 16 * k, 16)]
        pltpu.async_copy(pooled_hbm.at[gidx], rows_buf, sem)
        out_ref[0] = 1
        return cv - G

    def _scan_buf(db, sb, cvec):
        def _blk(b, cv):
            for u in range(BLK):
                e0 = (b * BLK + u) * 16
                d = db[pl.ds(e0, 16)]
                s = sb[pl.ds(e0, 16)]
                m = (d >= lo) & (d < hi)
                cumv = plsc.cumsum(jnp.where(m, 1, 0))
                pos = cv + cumv - 1
                plsc.store_scatter(pend_src, [pos], s, mask=m)
                plsc.store_scatter(pend_ld, [pos], d - lo, mask=m)
                cv = cv + plsc.all_reduce_population_count(m)
            return lax.cond(cv[0] >= G, _flush_pend, lambda x: x, cv)

        return lax.fori_loop(0, VECS // BLK, _blk, cvec)

    def _start_chunk(ci, db, sb):
        pltpu.async_copy(dst_hbm.at[pl.ds(ci * CH, CH)], db, semd)
        pltpu.async_copy(src_hbm.at[pl.ds(ci * CH, CH)], sb, sems)

    def _wait_chunk(ci, db, sb):
        pltpu.make_async_copy(dst_hbm.at[pl.ds(ci * CH, CH)], db, semd).wait()
        pltpu.make_async_copy(src_hbm.at[pl.ds(ci * CH, CH)], sb, sems).wait()

    _start_chunk(0, dst_buf, src_buf)

    def _outer(g, cv):
        for u, (db, sb) in ((0, (dst_buf, src_buf)), (1, (dst_buf2, src_buf2))):
            ci = 2 * g + u
            _wait_chunk(ci, db, sb)
            nb = (dst_buf2, src_buf2) if u == 0 else (dst_buf, src_buf)

            @pl.when(ci + 1 < NCH)
            def _pref():
                _start_chunk(ci + 1, nb[0], nb[1])
            cv = _scan_buf(db, sb, cv)
        return cv

    cvec = lax.fori_loop(0, NCH // 2, _outer, jnp.zeros((16,), jnp.int32))

    _wait_and_accum_inflight()

    c = cvec[0]
    pend_src[pl.ds(c, 16)] = jnp.zeros((16,), jnp.int32)
    pend_ld[pl.ds(c, 16)] = jnp.full((16,), TRASH, jnp.int32)
    nv = (c + 15) // 16

    def _drain(i, _):
        pltpu.async_copy(
            pooled_hbm.at[pend_src.at[pl.ds(i * 16, 16)]],
            rows_buf.at[pl.ds(0, 16)], sem
        ).wait()

        _accum_group(rows_buf, 0, pend_ld, i * 16)
        return 0
    lax.fori_loop(0, nv, _drain, 0)

    pltpu.sync_copy(acc.at[pl.ds(0, RPT)], out_hbm.at[pl.ds(lo, RPT)])


@functools.partial(
    pl.kernel,
    out_type=jax.ShapeDtypeStruct((OUT_ROWS, D_FEAT), jnp.float32),
    mesh=plsc.VectorSubcoreMesh(core_axis_name="c", subcore_axis_name="s"),
    compiler_params=pltpu.CompilerParams(needs_layout_passes=False),
    scratch_types=[
        pltpu.VMEM((RPT + 1, D_FEAT), jnp.float32),
        pltpu.VMEM((CH,), jnp.int32),
        pltpu.VMEM((CH,), jnp.int32),
        pltpu.VMEM((CH,), jnp.int32),
        pltpu.VMEM((CH,), jnp.int32),
        pltpu.VMEM((PEND,), jnp.int32),
        pltpu.VMEM((PEND,), jnp.int32),
        pltpu.VMEM((G,), jnp.int32),
        pltpu.VMEM((G,), jnp.int32),
        pltpu.VMEM((G, D_FEAT), jnp.float32),
        pltpu.SMEM((1,), jnp.int32),
        pltpu.SemaphoreType.DMA,
        pltpu.SemaphoreType.DMA,
        pltpu.SemaphoreType.DMA,
    ],
)
def _sc_segmax(pooled_hbm, src_hbm, dst_hbm, out_hbm,
               acc, dst_buf, src_buf, dst_buf2, src_buf2,
               pend_src, pend_ld, gidx, gld, rows_buf, out_ref, sem, semd, sems):
    _sc_body(pooled_hbm, src_hbm, dst_hbm, out_hbm,
             acc, dst_buf, src_buf, dst_buf2, src_buf2,
             pend_src, pend_ld, gidx, gld, rows_buf, out_ref, sem, semd, sems)


def kernel(x, edge_index, W_pool, b_pool, W_self, W_neigh, bias, W_lin, b_lin):
    src = edge_index[0]
    dst = edge_index[1]
    pooled, xs = _tc_pool(x, W_pool.T, b_pool.reshape(1, -1), W_self.T)
    hn_pad = _sc_segmax(pooled, src, dst)
    return _tc_head(xs, hn_pad, W_neigh.T, bias.reshape(1, -1),
                    W_lin.T, b_lin.reshape(1, -1))


# speedup vs baseline: 4.1209x; 1.0053x over previous
import functools

import jax
import jax.numpy as jnp
from jax import lax
from jax.experimental import pallas as pl
from jax.experimental.pallas import tpu as pltpu
from jax.experimental.pallas import tpu_sc as plsc

N_NODES = 10000
N_EDGES = 320000
D_FEAT = 128
N_CLASSES = 16

NC = 2
NS = 16
NW = NC * NS

RPT = 320
OUT_ROWS = NW * RPT
TRASH = RPT
CH = 6400
NCH = N_EDGES // CH
VECS = CH // 16
BLK = 8
G = 128
PEND = 272

ROW_BLK = 1000


def _tc_pool_body(x_ref, wpT_ref, bp_ref, wsT_ref, pooled_ref, xs_ref):
    xb = x_ref[...]
    p = jnp.dot(xb, wpT_ref[...], preferred_element_type=jnp.float32)
    pooled_ref[...] = jnp.maximum(p + bp_ref[...], 0.0)
    xs_ref[...] = jnp.dot(xb, wsT_ref[...], preferred_element_type=jnp.float32)


def _tc_pool(x, wpT, bp, wsT):
    grid = (N_NODES // ROW_BLK,)
    return pl.pallas_call(
        _tc_pool_body,
        grid=grid,
        in_specs=[
            pl.BlockSpec((ROW_BLK, D_FEAT), lambda i: (i, 0)),
            pl.BlockSpec((D_FEAT, D_FEAT), lambda i: (0, 0)),
            pl.BlockSpec((1, D_FEAT), lambda i: (0, 0)),
            pl.BlockSpec((D_FEAT, D_FEAT), lambda i: (0, 0)),
        ],
        out_specs=[
            pl.BlockSpec((ROW_BLK, D_FEAT), lambda i: (i, 0)),
            pl.BlockSpec((ROW_BLK, D_FEAT), lambda i: (i, 0)),
        ],
        out_shape=[
            jax.ShapeDtypeStruct((N_NODES, D_FEAT), jnp.float32),
            jax.ShapeDtypeStruct((N_NODES, D_FEAT), jnp.float32),
        ],
    )(x, wpT, bp, wsT)


def _tc_head_body(xs_ref, hn_ref, wnT_ref, b_ref, wlT_ref, bl_ref, out_ref):
    h = xs_ref[...] + jnp.dot(hn_ref[...], wnT_ref[...],
                              preferred_element_type=jnp.float32) + b_ref[...]
    h = jnp.where(h >= 0.0, h, 0.01 * h)
    z = jnp.dot(h, wlT_ref[...], preferred_element_type=jnp.float32) + bl_ref[...]
    out_ref[...] = jax.nn.sigmoid(z)


def _tc_head(xs, hn, wnT, b, wlT, bl):
    grid = (N_NODES // ROW_BLK,)
    return pl.pallas_call(
        _tc_head_body,
        grid=grid,
        in_specs=[
            pl.BlockSpec((ROW_BLK, D_FEAT), lambda i: (i, 0)),
            pl.BlockSpec((ROW_BLK, D_FEAT), lambda i: (i, 0)),
            pl.BlockSpec((D_FEAT, D_FEAT), lambda i: (0, 0)),
            pl.BlockSpec((1, D_FEAT), lambda i: (0, 0)),
            pl.BlockSpec((D_FEAT, N_CLASSES), lambda i: (0, 0)),
            pl.BlockSpec((1, N_CLASSES), lambda i: (0, 0)),
        ],
        out_specs=pl.BlockSpec((ROW_BLK, N_CLASSES), lambda i: (i, 0)),
        out_shape=jax.ShapeDtypeStruct((N_NODES, N_CLASSES), jnp.float32),
    )(xs, hn, wnT, b, wlT, bl)


def _sc_body(pooled_hbm, src_hbm, dst_hbm, out_hbm,
             acc, dst_buf, src_buf, dst_buf2, src_buf2,
             pend_src, pend_ld, gidx, gld, rows_buf, out_ref, sem, semd, sems):
    wid = lax.axis_index("s") * NC + lax.axis_index("c")
    lo = wid * RPT
    hi = lo + RPT

    zero16 = jnp.zeros((16,), jnp.float32)

    def _zrow(r, _):
        for f in range(8):
            acc[r, pl.ds(16 * f, 16)] = zero16
        return 0
    lax.fori_loop(0, RPT + 1, _zrow, 0)
    out_ref[0] = 0

    def _accum_group(rows_ref, row_base, ld_ref, ld_base):
        ldv = ld_ref[pl.ds(ld_base, 16)]
        for i in range(16):
            ld = ldv[i]
            for f in range(8):
                sl = pl.ds(16 * f, 16)
                acc[ld, sl] = jnp.maximum(acc[ld, sl], rows_ref[row_base + i, sl])

    def _wait_and_accum_inflight():
        @pl.when(out_ref[0] == 1)
        def _():
            pltpu.make_async_copy(pooled_hbm.at[gidx], rows_buf, sem).wait()

            def _agrp(g, _):
                _accum_group(rows_buf, g * 16, gld, g * 16)
                return 0
            lax.fori_loop(0, G // 16, _agrp, 0)
            out_ref[0] = 0

    def _flush_pend(cv):
        _wait_and_accum_inflight()
        for k in range(G // 16):
            gidx[pl.ds(16 * k, 16)] = pend_src[pl.ds(16 * k, 16)]
            gld[pl.ds(16 * k, 16)] = pend_ld[pl.ds(16 * k, 16)]
        for k in range(BLK):
            pend_src[pl.ds(16 * k, 16)] = pend_src[pl.ds(G + 16 * k, 16)]
            pend_ld[pl.ds(16 * k, 16)] = pend_ld[pl.ds(G + 16 * k, 16)]
        pltpu.async_copy(pooled_hbm.at[gidx], rows_buf, sem)
        out_ref[0] = 1
        return cv - G

    def _scan_buf(db, sb, cvec):
        def _blk(b, cv):
            for u in range(BLK):
                e0 = (b * BLK + u) * 16
                d = db[pl.ds(e0, 16)]
                s = sb[pl.ds(e0, 16)]
                m = (d >= lo) & (d < hi)
                cumv = plsc.cumsum(jnp.where(m, 1, 0))
                pos = cv + cumv - 1
                plsc.store_scatter(pend_src, [pos], s, mask=m)
                plsc.store_scatter(pend_ld, [pos], d - lo, mask=m)
                cv = cv + plsc.all_reduce_population_count(m)
            return lax.cond(cv[0] >= G, _flush_pend, lambda x: x, cv)

        return lax.fori_loop(0, VECS // BLK, _blk, cvec)

    def _start_chunk(ci, db, sb):
        pltpu.async_copy(dst_hbm.at[pl.ds(ci * CH, CH)], db, semd)
        pltpu.async_copy(src_hbm.at[pl.ds(ci * CH, CH)], sb, sems)

    def _wait_chunk(ci, db, sb):
        pltpu.make_async_copy(dst_hbm.at[pl.ds(ci * CH, CH)], db, semd).wait()
        pltpu.make_async_copy(src_hbm.at[pl.ds(ci * CH, CH)], sb, sems).wait()

    _start_chunk(0, dst_buf, src_buf)

    def _outer(g, cv):
        for u, (db, sb) in ((0, (dst_buf, src_buf)), (1, (dst_buf2, src_buf2))):
            ci = 2 * g + u
            _wait_chunk(ci, db, sb)
            nb = (dst_buf2, src_buf2) if u == 0 else (dst_buf, src_buf)

            @pl.when(ci + 1 < NCH)
            def _pref():
                _start_chunk(ci + 1, nb[0], nb[1])
            cv = _scan_buf(db, sb, cv)
        return cv

    cvec = lax.fori_loop(0, NCH // 2, _outer, jnp.zeros((16,), jnp.int32))

    _wait_and_accum_inflight()

    c = cvec[0]
    pend_src[pl.ds(c, 16)] = jnp.zeros((16,), jnp.int32)
    pend_ld[pl.ds(c, 16)] = jnp.full((16,), TRASH, jnp.int32)
    nv = (c + 15) // 16

    def _drain(i, _):
        pltpu.async_copy(
            pooled_hbm.at[pend_src.at[pl.ds(i * 16, 16)]],
            rows_buf.at[pl.ds(0, 16)], sem
        ).wait()

        _accum_group(rows_buf, 0, pend_ld, i * 16)
        return 0
    lax.fori_loop(0, nv, _drain, 0)

    pltpu.sync_copy(acc.at[pl.ds(0, RPT)], out_hbm.at[pl.ds(lo, RPT)])


@functools.partial(
    pl.kernel,
    out_type=jax.ShapeDtypeStruct((OUT_ROWS, D_FEAT), jnp.float32),
    mesh=plsc.VectorSubcoreMesh(core_axis_name="c", subcore_axis_name="s"),
    compiler_params=pltpu.CompilerParams(needs_layout_passes=False),
    scratch_types=[
        pltpu.VMEM((RPT + 1, D_FEAT), jnp.float32),
        pltpu.VMEM((CH,), jnp.int32),
        pltpu.VMEM((CH,), jnp.int32),
        pltpu.VMEM((CH,), jnp.int32),
        pltpu.VMEM((CH,), jnp.int32),
        pltpu.VMEM((PEND,), jnp.int32),
        pltpu.VMEM((PEND,), jnp.int32),
        pltpu.VMEM((G,), jnp.int32),
        pltpu.VMEM((G,), jnp.int32),
        pltpu.VMEM((G, D_FEAT), jnp.float32),
        pltpu.SMEM((1,), jnp.int32),
        pltpu.SemaphoreType.DMA,
        pltpu.SemaphoreType.DMA,
        pltpu.SemaphoreType.DMA,
    ],
)
def _sc_segmax(pooled_hbm, src_hbm, dst_hbm, out_hbm,
               acc, dst_buf, src_buf, dst_buf2, src_buf2,
               pend_src, pend_ld, gidx, gld, rows_buf, out_ref, sem, semd, sems):
    _sc_body(pooled_hbm, src_hbm, dst_hbm, out_hbm,
             acc, dst_buf, src_buf, dst_buf2, src_buf2,
             pend_src, pend_ld, gidx, gld, rows_buf, out_ref, sem, semd, sems)


def kernel(x, edge_index, W_pool, b_pool, W_self, W_neigh, bias, W_lin, b_lin):
    src = edge_index[0]
    dst = edge_index[1]
    pooled, xs = _tc_pool(x, W_pool.T, b_pool.reshape(1, -1), W_self.T)
    hn_pad = _sc_segmax(pooled, src, dst)
    hn = hn_pad[:N_NODES]
    return _tc_head(xs, hn, W_neigh.T, bias.reshape(1, -1),
                    W_lin.T, b_lin.reshape(1, -1))
